# baseline (speedup 1.0000x reference)
import functools

import jax
import jax.numpy as jnp
from jax import lax
from jax.experimental import pallas as pl
from jax.experimental import pallas as pl_
from jax.experimental.pallas import tpu as pltpu
from jax.experimental.pallas import tpu_sc as plsc

N_NODES = 100000
HIDDEN = 128
NUM_SEGMENTS = 1024
CHUNK = 128
NUM_WORKERS = 32
K_UNIF = 24
NUM_UNIF = K_UNIF * NUM_WORKERS
NUM_FULL = N_NODES // CHUNK
NUM_EXTRA = NUM_FULL - NUM_UNIF
TAIL = N_NODES - NUM_FULL * CHUNK
NBUF = 2
CPB = 3
NBLK = K_UNIF // CPB
ROWS_PER_SUBCORE = NUM_SEGMENTS // 16


def _sc_pool(x_hbm, ids2d_hbm, ids1d_hbm, zeros_hbm, acc_hbm,
             ids_all, extra_ids_v, tail_ids_v, xbuf, shared_acc,
             load_sems, scat_sems):
    cid = lax.axis_index("c")
    sid = lax.axis_index("s")
    wid = sid * 2 + cid

    pltpu.sync_copy(zeros_hbm, shared_acc.at[pl.ds(sid * ROWS_PER_SUBCORE,
                                                   ROWS_PER_SUBCORE)])
    plsc.subcore_barrier()

    start = wid * K_UNIF
    pltpu.sync_copy(ids2d_hbm.at[pl.ds(start, K_UNIF)], ids_all)

    def load(blk, b):
        return pltpu.async_copy(
            x_hbm.at[pl.ds((start + blk * CPB) * CHUNK, CPB * CHUNK)],
            xbuf.at[b], load_sems.at[b])

    lh = {blk: load(blk, blk % NBUF) for blk in range(NBUF)}
    sh = {}
    for blk in range(NBLK):
        b = blk % NBUF
        lh[blk].wait()
        for j in range(CPB):
            k = blk * CPB + j
            sh[k] = pltpu.async_copy(
                xbuf.at[b].at[pl.ds(j * CHUNK, CHUNK)],
                shared_acc.at[ids_all.at[k]], scat_sems.at[b], add=True)
        if blk + NBUF < NBLK:
            for j in range(CPB):
                sh[blk * CPB + j].wait()
            lh[blk + NBUF] = load(blk + NBUF, b)
    for k in range((NBLK - NBUF) * CPB, K_UNIF):
        sh[k].wait()

    @pl.when(wid < NUM_EXTRA)
    def _():
        base = (NUM_UNIF + wid) * CHUNK
        pltpu.sync_copy(ids2d_hbm.at[pl.ds(NUM_UNIF + wid, 1)], extra_ids_v)
        pltpu.sync_copy(x_hbm.at[pl.ds(base, CHUNK)],
                        xbuf.at[0].at[pl.ds(0, CHUNK)])
        pltpu.sync_copy(xbuf.at[0].at[pl.ds(0, CHUNK)],
                        shared_acc.at[extra_ids_v.at[0]], add=True)

    @pl.when(wid == NUM_WORKERS - 1)
    def _():
        base = NUM_FULL * CHUNK
        pltpu.sync_copy(ids1d_hbm.at[pl.ds(base, TAIL)], tail_ids_v.at[0])
        pltpu.sync_copy(x_hbm.at[pl.ds(base, TAIL)], xbuf.at[0].at[pl.ds(0, TAIL)])
        pltpu.sync_copy(xbuf.at[0].at[pl.ds(0, TAIL)],
                        shared_acc.at[tail_ids_v.at[0]], add=True)

    plsc.subcore_barrier()

    sl = pl.ds(sid * ROWS_PER_SUBCORE, ROWS_PER_SUBCORE)
    pltpu.sync_copy(shared_acc.at[sl], acc_hbm.at[cid].at[sl])


def _combine(acc_ref, out_ref):
    out_ref[...] = acc_ref[0] + acc_ref[1]


def kernel(node_states, segment_ids):
    ids32 = segment_ids.astype(jnp.int32)
    ids2d = ids32[:NUM_FULL * CHUNK].reshape(NUM_FULL, CHUNK)
    zeros = jnp.zeros((ROWS_PER_SUBCORE, HIDDEN), jnp.float32)

    sc_pool = pl.kernel(
        _sc_pool,
        out_type=jax.ShapeDtypeStruct((2, NUM_SEGMENTS, HIDDEN), jnp.float32),
        mesh=plsc.VectorSubcoreMesh(core_axis_name="c", subcore_axis_name="s"),
        scratch_types=[
            pltpu.VMEM((K_UNIF, CHUNK), jnp.int32),
            pltpu.VMEM((1, CHUNK), jnp.int32),
            pltpu.VMEM((1, TAIL), jnp.int32),
            pltpu.VMEM((NBUF, CPB * CHUNK, HIDDEN), jnp.float32),
            pltpu.VMEM_SHARED((NUM_SEGMENTS, HIDDEN), jnp.float32),
            pltpu.SemaphoreType.DMA((NBUF,)),
            pltpu.SemaphoreType.DMA((NBUF,)),
        ],
    )
    acc = sc_pool(node_states, ids2d, ids32, zeros)

    return pl.pallas_call(
        _combine,
        out_shape=jax.ShapeDtypeStruct((NUM_SEGMENTS, HIDDEN), jnp.float32),
    )(acc)


# speedup vs baseline: 1.0992x; 1.0992x over previous
import functools

import jax
import jax.numpy as jnp
from jax import lax
from jax.experimental import pallas as pl
from jax.experimental.pallas import tpu as pltpu
from jax.experimental.pallas import tpu_sc as plsc

N_NODES = 100000
HIDDEN = 128
NUM_SEGMENTS = 1024

CHUNK = 128
NUM_WORKERS = 32
K_SC = 10
N_SC = NUM_WORKERS * K_SC * CHUNK
NBUF = 4
ROWS_PER_SUBCORE = NUM_SEGMENTS // 16

BLOCK_R = 2048
N_TC = N_NODES - N_SC
TC_BLOCKS = (N_TC + BLOCK_R - 1) // BLOCK_R
TC_BLOCK0 = N_SC // BLOCK_R
WIN = 128


def _sc_pool(x_hbm, ids2d_hbm, zeros_hbm, acc_hbm,
             ids_all, xbuf, shared_acc, load_sems, scat_sems):
    cid = lax.axis_index("c")
    sid = lax.axis_index("s")
    wid = sid * 2 + cid

    pltpu.sync_copy(zeros_hbm, shared_acc.at[pl.ds(sid * ROWS_PER_SUBCORE,
                                                   ROWS_PER_SUBCORE)])
    plsc.subcore_barrier()

    start = wid * K_SC
    pltpu.sync_copy(ids2d_hbm.at[wid], ids_all)

    def load(k, b):
        return pltpu.async_copy(
            x_hbm.at[pl.ds((start + k) * CHUNK, CHUNK)], xbuf.at[b],
            load_sems.at[b])

    lh = {k: load(k, k % NBUF) for k in range(NBUF)}
    sh = {}
    for k in range(K_SC):
        b = k % NBUF
        lh[k].wait()
        sh[k] = pltpu.async_copy(xbuf.at[b], shared_acc.at[ids_all.at[k]],
                                 scat_sems.at[b], add=True)
        if k + NBUF < K_SC:
            sh[k].wait()
            lh[k + NBUF] = load(k + NBUF, b)
    for k in range(max(K_SC - NBUF, 0), K_SC):
        sh[k].wait()

    plsc.subcore_barrier()

    sl = pl.ds(sid * ROWS_PER_SUBCORE, ROWS_PER_SUBCORE)
    pltpu.sync_copy(shared_acc.at[sl], acc_hbm.at[cid].at[sl])


def _tc_pool(ids_ref, x_ref, out_ref):
    i = pl.program_id(0)

    @pl.when(i == 0)
    def _():
        out_ref[...] = jnp.zeros_like(out_ref)

    ids = ids_ref[0, 0, :]
    row = i * BLOCK_R + jax.lax.broadcasted_iota(jnp.int32, (BLOCK_R, HIDDEN), 0)
    x = jnp.where(row < N_TC, x_ref[...], 0.0).astype(jnp.bfloat16)

    c0 = ids_ref[0, 0, 0] // WIN
    c1 = jnp.minimum(ids_ref[0, 0, BLOCK_R - 1], NUM_SEGMENTS - 1) // WIN

    def body(c, _):
        seg = c * WIN + jax.lax.broadcasted_iota(jnp.int32, (WIN, BLOCK_R), 0)
        one_hot = (seg == ids[None, :]).astype(jnp.bfloat16)
        out_ref[c, :, :] += jnp.dot(
            one_hot, x, preferred_element_type=jnp.float32)
        return 0

    lax.fori_loop(c0, c1 + 1, body, 0)


def _combine(acc_ref, tc_ref, out_ref):
    out_ref[...] = acc_ref[0] + acc_ref[1] + tc_ref[...]


def kernel(node_states, segment_ids):
    ids32 = segment_ids.astype(jnp.int32)
    ids2d = ids32[:N_SC].reshape(NUM_WORKERS, K_SC, CHUNK)
    zeros = jnp.zeros((ROWS_PER_SUBCORE, HIDDEN), jnp.float32)

    n_tc_pad = TC_BLOCKS * BLOCK_R
    ids_tc = jnp.full((n_tc_pad,), 2047, jnp.int32)
    ids_tc = ids_tc.at[:N_TC].set(ids32[N_SC:])
    ids_tc = ids_tc.reshape(TC_BLOCKS, 1, BLOCK_R)

    sc_pool = pl.kernel(
        _sc_pool,
        out_type=jax.ShapeDtypeStruct((2, NUM_SEGMENTS, HIDDEN), jnp.float32),
        mesh=plsc.VectorSubcoreMesh(core_axis_name="c", subcore_axis_name="s"),
        scratch_types=[
            pltpu.VMEM((K_SC, CHUNK), jnp.int32),
            pltpu.VMEM((NBUF, CHUNK, HIDDEN), jnp.float32),
            pltpu.VMEM_SHARED((NUM_SEGMENTS, HIDDEN), jnp.float32),
            pltpu.SemaphoreType.DMA((NBUF,)),
            pltpu.SemaphoreType.DMA((NBUF,)),
        ],
    )
    acc = sc_pool(node_states, ids2d, zeros)

    tc_out = pl.pallas_call(
        _tc_pool,
        grid=(TC_BLOCKS,),
        in_specs=[
            pl.BlockSpec((1, 1, BLOCK_R), lambda i: (i, 0, 0)),
            pl.BlockSpec((BLOCK_R, HIDDEN), lambda i: (i + TC_BLOCK0, 0)),
        ],
        out_specs=pl.BlockSpec((NUM_SEGMENTS // WIN, WIN, HIDDEN),
                               lambda i: (0, 0, 0)),
        out_shape=jax.ShapeDtypeStruct((NUM_SEGMENTS // WIN, WIN, HIDDEN),
                                       jnp.float32),
        compiler_params=pltpu.CompilerParams(
            dimension_semantics=("arbitrary",),
        ),
    )(ids_tc, node_states)
    tc_out = tc_out.reshape(NUM_SEGMENTS, HIDDEN)

    return pl.pallas_call(
        _combine,
        out_shape=jax.ShapeDtypeStruct((NUM_SEGMENTS, HIDDEN), jnp.float32),
    )(acc, tc_out)
